# Initial kernel scaffold; baseline (speedup 1.0000x reference)
import jax
import jax.numpy as jnp
from jax.experimental import pallas as pl


def kernel(atoms, degrees, atom_table, degree_table):
    raise NotImplementedError("write your pallas kernel here")


# speedup vs baseline: 5.7125x; 5.7125x over previous
import functools

import jax
import jax.numpy as jnp
from jax import lax
from jax.experimental import pallas as pl
from jax.experimental.pallas import tpu as pltpu
from jax.experimental.pallas import tpu_sc as plsc

NC = 2
NS = 16
NW = NC * NS

CH = 128
D = 64
NBUF = 4


def _sc_kernel(n_chunks_per_worker, atoms2, degrees2, atab, dtab, out3,
               ia, id_, ra, rd, sI, sA, sB, sO):
    wid = lax.axis_index("s") * NC + lax.axis_index("c")
    base = wid * n_chunks_per_worker

    def idx_fetch(slot, t):
        pltpu.async_copy(atoms2.at[base + t], ia[slot], sI[slot])
        pltpu.async_copy(degrees2.at[base + t], id_[slot], sI[slot])

    def idx_wait(slot, t):
        pltpu.make_async_copy(atoms2.at[base + t], ia[slot], sI[slot]).wait()
        pltpu.make_async_copy(degrees2.at[base + t], id_[slot], sI[slot]).wait()

    def gather_issue(slot):
        pltpu.async_copy(atab.at[ia[slot]], ra[slot], sA[slot])
        pltpu.async_copy(dtab.at[id_[slot]], rd[slot], sB[slot])

    def gather_wait(slot):
        pltpu.make_async_copy(atab.at[ia[slot]], ra[slot], sA[slot]).wait()
        pltpu.make_async_copy(dtab.at[id_[slot]], rd[slot], sB[slot]).wait()

    def scatter_issue(slot, t):
        pltpu.async_copy(ra[slot], out3.at[base + t], sO[slot])

    def scatter_wait(slot, t):
        pltpu.make_async_copy(ra[slot], out3.at[base + t], sO[slot]).wait()

    for b in range(NBUF):
        idx_fetch(b, b)
    for b in range(2):
        idx_wait(b, b)
        gather_issue(b)

    def group_body(g, carry):
        for b in range(NBUF):
            t = g * NBUF + b
            b2 = (b + 2) % NBUF

            @pl.when(t >= 2)
            def _():
                scatter_wait(b2, t - 2)

            @pl.when(t <= n_chunks_per_worker - 3)
            def _():
                idx_wait(b2, t + 2)
                gather_issue(b2)

            gather_wait(b)

            @pl.when(t <= n_chunks_per_worker - 5)
            def _():
                idx_fetch(b, t + 4)

            def add_row(i, _c):
                for j in range(D // 16):
                    sl = pl.ds(j * 16, 16)
                    ra[b][i, sl] = ra[b][i, sl] + rd[b][i, sl]
                return _c

            lax.fori_loop(0, CH, add_row, 0)

            scatter_issue(b, t)
        return carry

    lax.fori_loop(0, n_chunks_per_worker // NBUF, group_body, 0)

    for t in (n_chunks_per_worker - 2, n_chunks_per_worker - 1):
        scatter_wait(t % NBUF, t)


def kernel(atoms, degrees, atom_table, degree_table):
    n_rows, n_cols = atoms.shape
    total = n_rows * n_cols
    n_chunks = total // CH
    n_chunks_per_worker = n_chunks // NW
    atoms2 = atoms.reshape(n_chunks, CH).astype(jnp.int32)
    degrees2 = degrees.reshape(n_chunks, CH).astype(jnp.int32)

    mesh = plsc.VectorSubcoreMesh(core_axis_name="c", subcore_axis_name="s",
                                  num_cores=NC, num_subcores=NS)
    f = pl.kernel(
        functools.partial(_sc_kernel, n_chunks_per_worker),
        out_type=jax.ShapeDtypeStruct((n_chunks, CH, D), jnp.float32),
        mesh=mesh,
        compiler_params=pltpu.CompilerParams(use_tc_tiling_on_sc=False),
        scratch_types=(
            [pltpu.VMEM((CH,), jnp.int32) for _ in range(NBUF)],
            [pltpu.VMEM((CH,), jnp.int32) for _ in range(NBUF)],
            [pltpu.VMEM((CH, D), jnp.float32) for _ in range(NBUF)],
            [pltpu.VMEM((CH, D), jnp.float32) for _ in range(NBUF)],
            [pltpu.SemaphoreType.DMA for _ in range(NBUF)],
            [pltpu.SemaphoreType.DMA for _ in range(NBUF)],
            [pltpu.SemaphoreType.DMA for _ in range(NBUF)],
            [pltpu.SemaphoreType.DMA for _ in range(NBUF)],
        ),
    )
    out = f(atoms2, degrees2, atom_table, degree_table)
    return out.reshape(n_rows, n_cols, D)
